# baseline (speedup 1.0000x reference)
import functools
import jax
import jax.numpy as jnp
from jax import lax
from jax.experimental import pallas as pl
from jax.experimental.pallas import tpu as pltpu
from jax.experimental.pallas import tpu_sc as plsc

_B, _S, _D, _E, _FF, _K = 2, 2048, 1024, 8, 2048, 2
_T = _B * _S
_PAIRS = _T * _K
_TM = 256
_P = _PAIRS + _E * _TM
_NT3 = _P // _TM
_GT = 512

_NC, _NS, _L = 2, 16, 16
_NW = _NC * _NS
_PER_W = _PAIRS // _NW
_SUB = 64
_NSUB = _PER_W // _SUB
_TOK_W = _T // _NW


def _erf(x):
    p = 0.3275911
    s = jnp.sign(x)
    ax = jnp.abs(x)
    t = 1.0 / (1.0 + p * ax)
    poly = ((((1.061405429 * t - 1.453152027) * t + 1.421413741) * t
             - 0.284496736) * t + 0.254829592) * t
    return s * (1.0 - poly * jnp.exp(-ax * ax))


def _gelu(x):
    return 0.5 * x * (1.0 + _erf(x * 0.7071067811865476))


def _gelu_tanh(x):
    t = x * x
    inner = x * (0.7978845608028654 + 0.035677408136300125 * t)
    return 0.5 * x * (1.0 + jnp.tanh(inner))


def _gate_kernel(x_ref, gw_ref, gb_ref, idx_ref, w_ref, rank_ref, counts_ref,
                 aux_ref, carry_ref, psum_ref):
    i = pl.program_id(0)

    @pl.when(i == 0)
    def _():
        carry_ref[...] = jnp.zeros_like(carry_ref)
        psum_ref[...] = jnp.zeros_like(psum_ref)

    x = x_ref[...]
    logits = jax.lax.dot_general(x, gw_ref[...], (((1,), (0,)), ((), ())),
                                 preferred_element_type=jnp.float32)
    logits = logits + gb_ref[...][None, :]
    m = jnp.max(logits, axis=-1, keepdims=True)
    ex = jnp.exp(logits - m)
    probs = ex / jnp.sum(ex, axis=-1, keepdims=True)

    e_iota = jax.lax.broadcasted_iota(jnp.int32, probs.shape, 1)
    m1 = jnp.max(probs, axis=-1, keepdims=True)
    i1 = jnp.min(jnp.where(probs >= m1, e_iota, _E), axis=-1, keepdims=True)
    oh1 = e_iota == i1
    p2 = jnp.where(oh1, -1.0, probs)
    m2 = jnp.max(p2, axis=-1, keepdims=True)
    i2 = jnp.min(jnp.where(p2 >= m2, e_iota, _E), axis=-1, keepdims=True)
    oh2 = e_iota == i2

    denom = m1 + m2
    idx_ref[...] = jnp.concatenate([i1, i2], axis=1)
    w_ref[...] = jnp.concatenate([m1 / denom, m2 / denom], axis=1)

    ohc = jnp.where(oh1 | oh2, 1.0, 0.0)
    r_iota = jax.lax.broadcasted_iota(jnp.int32, (_GT, _GT), 0)
    c_iota = jax.lax.broadcasted_iota(jnp.int32, (_GT, _GT), 1)
    lstrict = jnp.where(c_iota < r_iota, 1.0, 0.0)
    c_excl = jax.lax.dot_general(lstrict, ohc, (((1,), (0,)), ((), ())),
                                 preferred_element_type=jnp.float32)
    c_tot = carry_ref[...] + c_excl
    r1 = jnp.sum(jnp.where(oh1, c_tot, 0.0), axis=-1, keepdims=True)
    r2 = jnp.sum(jnp.where(oh2, c_tot, 0.0), axis=-1, keepdims=True)
    rank_ref[...] = jnp.concatenate([r1, r2], axis=1).astype(jnp.int32)

    carry = carry_ref[...] + jnp.sum(ohc, axis=0, keepdims=True)
    psum = psum_ref[...] + jnp.sum(probs, axis=0, keepdims=True)
    carry_ref[...] = carry
    psum_ref[...] = psum

    counts_ref[...] = carry.astype(jnp.int32)
    aux = _E * jnp.sum((carry / (_T * _K)) * (psum / _T))
    aux_ref[...] = jnp.broadcast_to(aux, (1, 1))


def _slot_kernel(counts_ref, idx_ref, rank_ref, slot_ref):
    idx = idx_ref[...]
    slot = rank_ref[...]
    base = 0
    for e in range(_E):
        slot = jnp.where(idx == e, slot + base, slot)
        ce = counts_ref[0, e]
        base = base + ((ce + (_TM - 1)) // _TM) * _TM
    slot_ref[...] = slot


def _dispatch_body(slot_hbm, x_hbm, xg_hbm, tok_v, slot_v, rows_v,
                   sem1, sem2):
    wid = lax.axis_index("s") * _NC + lax.axis_index("c")
    base = wid * _PER_W
    pltpu.sync_copy(slot_hbm.at[pl.ds(wid * _NSUB, _NSUB)], slot_v)
    for j in range(_PER_W // _L):
        r, c = j // (_SUB // _L), (j % (_SUB // _L)) * _L
        pvec = base + j * _L + lax.iota(jnp.int32, _L)
        tok_v[r, pl.ds(c, _L)] = pvec & (_T - 1)
    for s in range(_NSUB):
        pltpu.async_copy(x_hbm.at[tok_v.at[s]], rows_v, sem1).wait()
        pltpu.async_copy(rows_v, xg_hbm.at[slot_v.at[s]], sem2).wait()


_NF = 2


def _ffn_kernel(eid_ref, xg_ref, w1_ref, b1_ref, w2_ref, b2_ref, yb_ref):
    f = pl.program_id(1)
    h = jax.lax.dot_general(xg_ref[...], w1_ref[0], (((1,), (0,)), ((), ())),
                            preferred_element_type=jnp.float32)
    h = _gelu_tanh(h + b1_ref[0])
    y = jax.lax.dot_general(h, w2_ref[0], (((1,), (0,)), ((), ())),
                            preferred_element_type=jnp.float32)

    @pl.when(f == 0)
    def _():
        yb_ref[...] = y + b2_ref[0]

    @pl.when(f > 0)
    def _():
        yb_ref[...] = yb_ref[...] + y


def _gather_body(slot_hbm, yb_hbm, y2_hbm, sidx_v, rows_v, sem1):
    wid = lax.axis_index("s") * _NC + lax.axis_index("c")
    tb = wid * _TOK_W
    for k in range(_K):
        pltpu.sync_copy(
            slot_hbm.at[pl.ds(k * (_T // _SUB) + wid * (_TOK_W // _SUB),
                              _TOK_W // _SUB)], sidx_v)
        for s in range(_TOK_W // _SUB):
            pltpu.async_copy(yb_hbm.at[sidx_v.at[s]], rows_v, sem1).wait()
            pltpu.sync_copy(rows_v, y2_hbm.at[k, pl.ds(tb + s * _SUB, _SUB)])


def _combine_kernel(y2_ref, w_ref, out_ref):
    w = w_ref[...]
    out_ref[...] = w[:, 0:1] * y2_ref[0] + w[:, 1:2] * y2_ref[1]


def kernel(x, gate_W, gate_b, W1, b1, W2, b2):
    x_flat = x.reshape(_T, _D)

    nt = _T // _GT
    idx, w, rank, counts, aux = pl.pallas_call(
        _gate_kernel,
        grid=(nt,),
        in_specs=[
            pl.BlockSpec((_GT, _D), lambda i: (i, 0)),
            pl.BlockSpec((_D, _E), lambda i: (0, 0)),
            pl.BlockSpec((_E,), lambda i: (0,)),
        ],
        out_specs=[
            pl.BlockSpec((_GT, _K), lambda i: (i, 0)),
            pl.BlockSpec((_GT, _K), lambda i: (i, 0)),
            pl.BlockSpec((_GT, _K), lambda i: (i, 0)),
            pl.BlockSpec((1, _E), lambda i: (0, 0)),
            pl.BlockSpec((1, 1), lambda i: (0, 0)),
        ],
        out_shape=[
            jax.ShapeDtypeStruct((_T, _K), jnp.int32),
            jax.ShapeDtypeStruct((_T, _K), jnp.float32),
            jax.ShapeDtypeStruct((_T, _K), jnp.int32),
            jax.ShapeDtypeStruct((1, _E), jnp.int32),
            jax.ShapeDtypeStruct((1, 1), jnp.float32),
        ],
        scratch_shapes=[
            pltpu.VMEM((1, _E), jnp.float32),
            pltpu.VMEM((1, _E), jnp.float32),
        ],
    )(x_flat, gate_W, gate_b)

    c = counts[0]
    cpad = ((c + (_TM - 1)) // _TM) * _TM
    base_incl = jnp.cumsum(cpad)
    tile_start = jnp.arange(_NT3, dtype=jnp.int32) * _TM
    eid = jnp.sum(tile_start[:, None] >= base_incl[None, :], axis=1)
    eid = jnp.minimum(eid, _E - 1).astype(jnp.int32)

    idx_km = idx.T.reshape(_PAIRS // _SUB, _SUB)
    rank_km = rank.T.reshape(_PAIRS // _SUB, _SUB)

    slot = pl.pallas_call(
        _slot_kernel,
        in_specs=[
            pl.BlockSpec(memory_space=pltpu.SMEM),
            pl.BlockSpec((_PAIRS // _SUB, _SUB), lambda: (0, 0)),
            pl.BlockSpec((_PAIRS // _SUB, _SUB), lambda: (0, 0)),
        ],
        out_specs=pl.BlockSpec((_PAIRS // _SUB, _SUB), lambda: (0, 0)),
        out_shape=jax.ShapeDtypeStruct((_PAIRS // _SUB, _SUB), jnp.int32),
    )(counts, idx_km, rank_km)

    mesh = plsc.VectorSubcoreMesh(core_axis_name="c", subcore_axis_name="s")
    xg = pl.kernel(
        _dispatch_body,
        out_type=jax.ShapeDtypeStruct((_P, _D), jnp.float32),
        mesh=mesh,
        scratch_types=[
            pltpu.VMEM((_NSUB, _SUB), jnp.int32),
            pltpu.VMEM((_NSUB, _SUB), jnp.int32),
            pltpu.VMEM((_SUB, _D), jnp.float32),
            pltpu.SemaphoreType.DMA,
            pltpu.SemaphoreType.DMA,
        ],
    )(slot, x_flat)

    yb = pl.pallas_call(
        _ffn_kernel,
        grid_spec=pltpu.PrefetchScalarGridSpec(
            num_scalar_prefetch=1,
            grid=(_NT3, _NF),
            in_specs=[
                pl.BlockSpec((_TM, _D), lambda i, f, eid: (i, 0)),
                pl.BlockSpec((1, _D, _FF // _NF),
                             lambda i, f, eid: (eid[i], 0, f)),
                pl.BlockSpec((1, 1, _FF // _NF),
                             lambda i, f, eid: (eid[i], 0, f)),
                pl.BlockSpec((1, _FF // _NF, _D),
                             lambda i, f, eid: (eid[i], f, 0)),
                pl.BlockSpec((1, 1, _D), lambda i, f, eid: (eid[i], 0, 0)),
            ],
            out_specs=pl.BlockSpec((_TM, _D), lambda i, f, eid: (i, 0)),
        ),
        out_shape=jax.ShapeDtypeStruct((_P, _D), jnp.float32),
    )(eid, xg, W1, b1.reshape(_E, 1, _FF), W2, b2.reshape(_E, 1, _D))

    y2 = pl.kernel(
        _gather_body,
        out_type=jax.ShapeDtypeStruct((_K, _T, _D), jnp.float32),
        mesh=plsc.VectorSubcoreMesh(core_axis_name="c", subcore_axis_name="s"),
        scratch_types=[
            pltpu.VMEM((_TOK_W // _SUB, _SUB), jnp.int32),
            pltpu.VMEM((_SUB, _D), jnp.float32),
            pltpu.SemaphoreType.DMA,
        ],
    )(slot, yb)

    out = pl.pallas_call(
        _combine_kernel,
        grid=(_T // _GT,),
        in_specs=[
            pl.BlockSpec((_K, _GT, _D), lambda i: (0, i, 0)),
            pl.BlockSpec((_GT, _K), lambda i: (i, 0)),
        ],
        out_specs=pl.BlockSpec((_GT, _D), lambda i: (i, 0)),
        out_shape=jax.ShapeDtypeStruct((_T, _D), jnp.float32),
    )(y2, w)

    return out.reshape(_B, _S, _D), aux[0, 0]


# speedup vs baseline: 1.3070x; 1.3070x over previous
import functools
import jax
import jax.numpy as jnp
from jax import lax
from jax.experimental import pallas as pl
from jax.experimental.pallas import tpu as pltpu
from jax.experimental.pallas import tpu_sc as plsc

_B, _S, _D, _E, _FF, _K = 2, 2048, 1024, 8, 2048, 2
_T = _B * _S
_PAIRS = _T * _K
_TM = 256
_P = _PAIRS + _E * _TM
_NT3 = _P // _TM
_GT = 512

_NC, _NS, _L = 2, 16, 16
_NW = _NC * _NS
_PER_W = _PAIRS // _NW
_SUB = 32
_NSUB = _PER_W // _SUB
_TOK_W = _T // _NW


def _erf(x):
    p = 0.3275911
    s = jnp.sign(x)
    ax = jnp.abs(x)
    t = 1.0 / (1.0 + p * ax)
    poly = ((((1.061405429 * t - 1.453152027) * t + 1.421413741) * t
             - 0.284496736) * t + 0.254829592) * t
    return s * (1.0 - poly * jnp.exp(-ax * ax))


def _gelu(x):
    return 0.5 * x * (1.0 + _erf(x * 0.7071067811865476))


def _gelu_tanh(x):
    t = x * x
    inner = x * (0.7978845608028654 + 0.035677408136300125 * t)
    return 0.5 * x * (1.0 + jnp.tanh(inner))


def _gate_kernel(x_ref, gw_ref, gb_ref, idx_ref, w_ref, rank_ref, counts_ref,
                 aux_ref, carry_ref, psum_ref):
    i = pl.program_id(0)

    @pl.when(i == 0)
    def _():
        carry_ref[...] = jnp.zeros_like(carry_ref)
        psum_ref[...] = jnp.zeros_like(psum_ref)

    x = x_ref[...]
    logits = jax.lax.dot_general(x, gw_ref[...], (((1,), (0,)), ((), ())),
                                 preferred_element_type=jnp.float32)
    logits = logits + gb_ref[...][None, :]
    m = jnp.max(logits, axis=-1, keepdims=True)
    ex = jnp.exp(logits - m)
    probs = ex / jnp.sum(ex, axis=-1, keepdims=True)

    e_iota = jax.lax.broadcasted_iota(jnp.int32, probs.shape, 1)
    m1 = jnp.max(probs, axis=-1, keepdims=True)
    i1 = jnp.min(jnp.where(probs >= m1, e_iota, _E), axis=-1, keepdims=True)
    oh1 = e_iota == i1
    p2 = jnp.where(oh1, -1.0, probs)
    m2 = jnp.max(p2, axis=-1, keepdims=True)
    i2 = jnp.min(jnp.where(p2 >= m2, e_iota, _E), axis=-1, keepdims=True)
    oh2 = e_iota == i2

    denom = m1 + m2
    idx_ref[...] = jnp.concatenate([i1, i2], axis=1)
    w_ref[...] = jnp.concatenate([m1 / denom, m2 / denom], axis=1)

    ohc = jnp.where(oh1 | oh2, 1.0, 0.0)
    r_iota = jax.lax.broadcasted_iota(jnp.int32, (_GT, _GT), 0)
    c_iota = jax.lax.broadcasted_iota(jnp.int32, (_GT, _GT), 1)
    lstrict = jnp.where(c_iota < r_iota, 1.0, 0.0)
    c_excl = jax.lax.dot_general(lstrict, ohc, (((1,), (0,)), ((), ())),
                                 preferred_element_type=jnp.float32)
    c_tot = carry_ref[...] + c_excl
    r1 = jnp.sum(jnp.where(oh1, c_tot, 0.0), axis=-1, keepdims=True)
    r2 = jnp.sum(jnp.where(oh2, c_tot, 0.0), axis=-1, keepdims=True)
    rank_ref[...] = jnp.concatenate([r1, r2], axis=1).astype(jnp.int32)

    carry = carry_ref[...] + jnp.sum(ohc, axis=0, keepdims=True)
    psum = psum_ref[...] + jnp.sum(probs, axis=0, keepdims=True)
    carry_ref[...] = carry
    psum_ref[...] = psum

    counts_ref[...] = carry.astype(jnp.int32)
    aux = _E * jnp.sum((carry / (_T * _K)) * (psum / _T))
    aux_ref[...] = jnp.broadcast_to(aux, (1, 1))


def _slot_kernel(counts_ref, idx_ref, rank_ref, slot_ref):
    idx = idx_ref[...]
    slot = rank_ref[...]
    base = 0
    for e in range(_E):
        slot = jnp.where(idx == e, slot + base, slot)
        ce = counts_ref[0, e]
        base = base + ((ce + (_TM - 1)) // _TM) * _TM
    slot_ref[...] = slot


def _dispatch_body(slot_hbm, x_hbm, xg_hbm, tok_v, slot_v, rows0_v, rows1_v,
                   gsem0, gsem1, ssem0, ssem1):
    wid = lax.axis_index("s") * _NC + lax.axis_index("c")
    base = wid * _PER_W
    pltpu.sync_copy(slot_hbm.at[pl.ds(wid * _NSUB, _NSUB)], slot_v)
    for j in range(_PER_W // _L):
        r, c = j // (_SUB // _L), (j % (_SUB // _L)) * _L
        pvec = base + j * _L + lax.iota(jnp.int32, _L)
        tok_v[r, pl.ds(c, _L)] = pvec & (_T - 1)
    rows = (rows0_v, rows1_v)
    gsem = (gsem0, gsem1)
    ssem = (ssem0, ssem1)
    g = [None] * _NSUB
    sc = [None] * _NSUB
    g[0] = pltpu.async_copy(x_hbm.at[tok_v.at[0]], rows0_v, gsem0)
    for s in range(_NSUB):
        b = s % 2
        g[s].wait()
        sc[s] = pltpu.async_copy(rows[b], xg_hbm.at[slot_v.at[s]], ssem[b])
        if s + 1 < _NSUB:
            if s >= 1:
                sc[s - 1].wait()
            g[s + 1] = pltpu.async_copy(x_hbm.at[tok_v.at[s + 1]],
                                        rows[1 - b], gsem[1 - b])
    sc[_NSUB - 1].wait()


def _ffn_kernel(eid_ref, xg_ref, w1_ref, b1_ref, w2_ref, b2_ref, yb_ref):
    h = jax.lax.dot_general(xg_ref[...], w1_ref[0], (((1,), (0,)), ((), ())),
                            preferred_element_type=jnp.float32)
    h = _gelu_tanh(h + b1_ref[0])
    y = jax.lax.dot_general(h, w2_ref[0], (((1,), (0,)), ((), ())),
                            preferred_element_type=jnp.float32)
    yb_ref[...] = y + b2_ref[0]


def _gather_body(slot_hbm, yb_hbm, y2_hbm, sidx_v, rows0_v, rows1_v,
                 gsem0, gsem1, wsem0, wsem1):
    wid = lax.axis_index("s") * _NC + lax.axis_index("c")
    tb = wid * _TOK_W
    nchunk = _TOK_W // _SUB
    for k in range(_K):
        pltpu.sync_copy(
            slot_hbm.at[pl.ds(k * (_T // _SUB) + wid * nchunk, nchunk)],
            sidx_v)
        rows = (rows0_v, rows1_v)
        gsem = (gsem0, gsem1)
        wsem = (wsem0, wsem1)
        g = [None] * nchunk
        wr = [None] * nchunk
        g[0] = pltpu.async_copy(yb_hbm.at[sidx_v.at[0]], rows0_v, gsem0)
        for s in range(nchunk):
            b = s % 2
            g[s].wait()
            wr[s] = pltpu.async_copy(
                rows[b], y2_hbm.at[k, pl.ds(tb + s * _SUB, _SUB)], wsem[b])
            if s + 1 < nchunk:
                if s >= 1:
                    wr[s - 1].wait()
                g[s + 1] = pltpu.async_copy(yb_hbm.at[sidx_v.at[s + 1]],
                                            rows[1 - b], gsem[1 - b])
        wr[nchunk - 1].wait()


def _combine_kernel(y2_ref, w_ref, out_ref):
    w = w_ref[...]
    out_ref[...] = w[:, 0:1] * y2_ref[0] + w[:, 1:2] * y2_ref[1]


def kernel(x, gate_W, gate_b, W1, b1, W2, b2):
    x_flat = x.reshape(_T, _D)

    nt = _T // _GT
    idx, w, rank, counts, aux = pl.pallas_call(
        _gate_kernel,
        grid=(nt,),
        in_specs=[
            pl.BlockSpec((_GT, _D), lambda i: (i, 0)),
            pl.BlockSpec((_D, _E), lambda i: (0, 0)),
            pl.BlockSpec((_E,), lambda i: (0,)),
        ],
        out_specs=[
            pl.BlockSpec((_GT, _K), lambda i: (i, 0)),
            pl.BlockSpec((_GT, _K), lambda i: (i, 0)),
            pl.BlockSpec((_GT, _K), lambda i: (i, 0)),
            pl.BlockSpec((1, _E), lambda i: (0, 0)),
            pl.BlockSpec((1, 1), lambda i: (0, 0)),
        ],
        out_shape=[
            jax.ShapeDtypeStruct((_T, _K), jnp.int32),
            jax.ShapeDtypeStruct((_T, _K), jnp.float32),
            jax.ShapeDtypeStruct((_T, _K), jnp.int32),
            jax.ShapeDtypeStruct((1, _E), jnp.int32),
            jax.ShapeDtypeStruct((1, 1), jnp.float32),
        ],
        scratch_shapes=[
            pltpu.VMEM((1, _E), jnp.float32),
            pltpu.VMEM((1, _E), jnp.float32),
        ],
    )(x_flat, gate_W, gate_b)

    c = counts[0]
    cpad = ((c + (_TM - 1)) // _TM) * _TM
    base_incl = jnp.cumsum(cpad)
    tile_start = jnp.arange(_NT3, dtype=jnp.int32) * _TM
    eid = jnp.sum(tile_start[:, None] >= base_incl[None, :], axis=1)
    eid = jnp.minimum(eid, _E - 1).astype(jnp.int32)

    idx_km = idx.T.reshape(_PAIRS // _SUB, _SUB)
    rank_km = rank.T.reshape(_PAIRS // _SUB, _SUB)

    slot = pl.pallas_call(
        _slot_kernel,
        in_specs=[
            pl.BlockSpec(memory_space=pltpu.SMEM),
            pl.BlockSpec((_PAIRS // _SUB, _SUB), lambda: (0, 0)),
            pl.BlockSpec((_PAIRS // _SUB, _SUB), lambda: (0, 0)),
        ],
        out_specs=pl.BlockSpec((_PAIRS // _SUB, _SUB), lambda: (0, 0)),
        out_shape=jax.ShapeDtypeStruct((_PAIRS // _SUB, _SUB), jnp.int32),
    )(counts, idx_km, rank_km)

    mesh = plsc.VectorSubcoreMesh(core_axis_name="c", subcore_axis_name="s")
    xg = pl.kernel(
        _dispatch_body,
        out_type=jax.ShapeDtypeStruct((_P, _D), jnp.float32),
        mesh=mesh,
        scratch_types=[
            pltpu.VMEM((_NSUB, _SUB), jnp.int32),
            pltpu.VMEM((_NSUB, _SUB), jnp.int32),
            pltpu.VMEM((_SUB, _D), jnp.float32),
            pltpu.VMEM((_SUB, _D), jnp.float32),
            pltpu.SemaphoreType.DMA,
            pltpu.SemaphoreType.DMA,
            pltpu.SemaphoreType.DMA,
            pltpu.SemaphoreType.DMA,
        ],
    )(slot, x_flat)

    yb = pl.pallas_call(
        _ffn_kernel,
        grid_spec=pltpu.PrefetchScalarGridSpec(
            num_scalar_prefetch=1,
            grid=(_NT3,),
            in_specs=[
                pl.BlockSpec((_TM, _D), lambda i, eid: (i, 0)),
                pl.BlockSpec((1, _D, _FF), lambda i, eid: (eid[i], 0, 0)),
                pl.BlockSpec((1, 1, _FF), lambda i, eid: (eid[i], 0, 0)),
                pl.BlockSpec((1, _FF, _D), lambda i, eid: (eid[i], 0, 0)),
                pl.BlockSpec((1, 1, _D), lambda i, eid: (eid[i], 0, 0)),
            ],
            out_specs=pl.BlockSpec((_TM, _D), lambda i, eid: (i, 0)),
        ),
        out_shape=jax.ShapeDtypeStruct((_P, _D), jnp.float32),
    )(eid, xg, W1, b1.reshape(_E, 1, _FF), W2, b2.reshape(_E, 1, _D))

    y2 = pl.kernel(
        _gather_body,
        out_type=jax.ShapeDtypeStruct((_K, _T, _D), jnp.float32),
        mesh=plsc.VectorSubcoreMesh(core_axis_name="c", subcore_axis_name="s"),
        scratch_types=[
            pltpu.VMEM((_TOK_W // _SUB, _SUB), jnp.int32),
            pltpu.VMEM((_SUB, _D), jnp.float32),
            pltpu.VMEM((_SUB, _D), jnp.float32),
            pltpu.SemaphoreType.DMA,
            pltpu.SemaphoreType.DMA,
            pltpu.SemaphoreType.DMA,
            pltpu.SemaphoreType.DMA,
        ],
    )(slot, yb)

    out = pl.pallas_call(
        _combine_kernel,
        grid=(_T // _GT,),
        in_specs=[
            pl.BlockSpec((_K, _GT, _D), lambda i: (0, i, 0)),
            pl.BlockSpec((_GT, _K), lambda i: (i, 0)),
        ],
        out_specs=pl.BlockSpec((_GT, _D), lambda i: (i, 0)),
        out_shape=jax.ShapeDtypeStruct((_T, _D), jnp.float32),
    )(y2, w)

    return out.reshape(_B, _S, _D), aux[0, 0]


# speedup vs baseline: 1.3832x; 1.0583x over previous
import functools
import jax
import jax.numpy as jnp
from jax import lax
from jax.experimental import pallas as pl
from jax.experimental.pallas import tpu as pltpu
from jax.experimental.pallas import tpu_sc as plsc

_B, _S, _D, _E, _FF, _K = 2, 2048, 1024, 8, 2048, 2
_T = _B * _S
_PAIRS = _T * _K
_TM = 256
_P = _PAIRS + _E * _TM
_NT3 = _P // _TM
_GT = 512

_NC, _NS, _L = 2, 16, 16
_NW = _NC * _NS
_PER_W = _PAIRS // _NW
_SUB = 64
_NSUB = _PER_W // _SUB
_TOK_W = _T // _NW


def _erf(x):
    p = 0.3275911
    s = jnp.sign(x)
    ax = jnp.abs(x)
    t = 1.0 / (1.0 + p * ax)
    poly = ((((1.061405429 * t - 1.453152027) * t + 1.421413741) * t
             - 0.284496736) * t + 0.254829592) * t
    return s * (1.0 - poly * jnp.exp(-ax * ax))


def _gelu(x):
    return 0.5 * x * (1.0 + _erf(x * 0.7071067811865476))


def _gelu_tanh(x):
    t = x * x
    inner = x * (0.7978845608028654 + 0.035677408136300125 * t)
    return 0.5 * x * (1.0 + jnp.tanh(inner))


def _gate_kernel(x_ref, gw_ref, gb_ref, idx_ref, w_ref, rank_ref, counts_ref,
                 aux_ref, carry_ref, psum_ref):
    i = pl.program_id(0)

    @pl.when(i == 0)
    def _():
        carry_ref[...] = jnp.zeros_like(carry_ref)
        psum_ref[...] = jnp.zeros_like(psum_ref)

    x = x_ref[...]
    logits = jax.lax.dot_general(x, gw_ref[...], (((1,), (0,)), ((), ())),
                                 preferred_element_type=jnp.float32)
    logits = logits + gb_ref[...][None, :]
    m = jnp.max(logits, axis=-1, keepdims=True)
    ex = jnp.exp(logits - m)
    probs = ex / jnp.sum(ex, axis=-1, keepdims=True)

    e_iota = jax.lax.broadcasted_iota(jnp.int32, probs.shape, 1)
    m1 = jnp.max(probs, axis=-1, keepdims=True)
    i1 = jnp.min(jnp.where(probs >= m1, e_iota, _E), axis=-1, keepdims=True)
    oh1 = e_iota == i1
    p2 = jnp.where(oh1, -1.0, probs)
    m2 = jnp.max(p2, axis=-1, keepdims=True)
    i2 = jnp.min(jnp.where(p2 >= m2, e_iota, _E), axis=-1, keepdims=True)
    oh2 = e_iota == i2

    denom = m1 + m2
    idx_ref[...] = jnp.concatenate([i1, i2], axis=1)
    w_ref[...] = jnp.concatenate([m1 / denom, m2 / denom], axis=1)

    ohc = jnp.where(oh1 | oh2, 1.0, 0.0)
    r_iota = jax.lax.broadcasted_iota(jnp.int32, (_GT, _GT), 0)
    c_iota = jax.lax.broadcasted_iota(jnp.int32, (_GT, _GT), 1)
    lstrict = jnp.where(c_iota < r_iota, 1.0, 0.0)
    c_excl = jax.lax.dot_general(lstrict, ohc, (((1,), (0,)), ((), ())),
                                 preferred_element_type=jnp.float32)
    c_tot = carry_ref[...] + c_excl
    r1 = jnp.sum(jnp.where(oh1, c_tot, 0.0), axis=-1, keepdims=True)
    r2 = jnp.sum(jnp.where(oh2, c_tot, 0.0), axis=-1, keepdims=True)
    rank_ref[...] = jnp.concatenate([r1, r2], axis=1).astype(jnp.int32)

    carry = carry_ref[...] + jnp.sum(ohc, axis=0, keepdims=True)
    psum = psum_ref[...] + jnp.sum(probs, axis=0, keepdims=True)
    carry_ref[...] = carry
    psum_ref[...] = psum

    counts_ref[...] = carry.astype(jnp.int32)
    aux = _E * jnp.sum((carry / (_T * _K)) * (psum / _T))
    aux_ref[...] = jnp.broadcast_to(aux, (1, 1))


def _slot_kernel(counts_ref, idx_ref, rank_ref, slot_ref):
    idx = idx_ref[...]
    slot = rank_ref[...]
    base = 0
    for e in range(_E):
        slot = jnp.where(idx == e, slot + base, slot)
        ce = counts_ref[0, e]
        base = base + ((ce + (_TM - 1)) // _TM) * _TM
    slot_ref[...] = slot


def _dispatch_body(slot_hbm, x_hbm, xg_hbm, tok_v, slot_v, rows_v,
                   sem1, sem2):
    wid = lax.axis_index("s") * _NC + lax.axis_index("c")
    base = wid * _PER_W
    pltpu.sync_copy(slot_hbm.at[pl.ds(wid * _NSUB, _NSUB)], slot_v)
    for j in range(_PER_W // _L):
        r, c = j // (_SUB // _L), (j % (_SUB // _L)) * _L
        pvec = base + j * _L + lax.iota(jnp.int32, _L)
        tok_v[r, pl.ds(c, _L)] = pvec & (_T - 1)
    for s in range(_NSUB):
        pltpu.async_copy(x_hbm.at[tok_v.at[s]], rows_v, sem1).wait()
        pltpu.async_copy(rows_v, xg_hbm.at[slot_v.at[s]], sem2).wait()


def _ffn_kernel(eid_ref, xg_ref, w1_ref, b1_ref, w2_ref, b2_ref, yb_ref):
    @pl.when(pl.program_id(0) < eid_ref[_NT3])
    def _():
        h = jax.lax.dot_general(xg_ref[...], w1_ref[0],
                                (((1,), (0,)), ((), ())),
                                preferred_element_type=jnp.float32)
        h = _gelu_tanh(h + b1_ref[0])
        y = jax.lax.dot_general(h, w2_ref[0], (((1,), (0,)), ((), ())),
                                preferred_element_type=jnp.float32)
        yb_ref[...] = y + b2_ref[0]


def _gather_body(slot_hbm, yb_hbm, y2_hbm, sidx_v, rows_v, sem1):
    wid = lax.axis_index("s") * _NC + lax.axis_index("c")
    tb = wid * _TOK_W
    for k in range(_K):
        pltpu.sync_copy(
            slot_hbm.at[pl.ds(k * (_T // _SUB) + wid * (_TOK_W // _SUB),
                              _TOK_W // _SUB)], sidx_v)
        for s in range(_TOK_W // _SUB):
            pltpu.async_copy(yb_hbm.at[sidx_v.at[s]], rows_v, sem1).wait()
            pltpu.sync_copy(rows_v, y2_hbm.at[k, pl.ds(tb + s * _SUB, _SUB)])


def _combine_kernel(y2_ref, w_ref, out_ref):
    w = w_ref[...]
    out_ref[...] = w[:, 0:1] * y2_ref[0] + w[:, 1:2] * y2_ref[1]


def kernel(x, gate_W, gate_b, W1, b1, W2, b2):
    x_flat = x.reshape(_T, _D)

    nt = _T // _GT
    idx, w, rank, counts, aux = pl.pallas_call(
        _gate_kernel,
        grid=(nt,),
        in_specs=[
            pl.BlockSpec((_GT, _D), lambda i: (i, 0)),
            pl.BlockSpec((_D, _E), lambda i: (0, 0)),
            pl.BlockSpec((_E,), lambda i: (0,)),
        ],
        out_specs=[
            pl.BlockSpec((_GT, _K), lambda i: (i, 0)),
            pl.BlockSpec((_GT, _K), lambda i: (i, 0)),
            pl.BlockSpec((_GT, _K), lambda i: (i, 0)),
            pl.BlockSpec((1, _E), lambda i: (0, 0)),
            pl.BlockSpec((1, 1), lambda i: (0, 0)),
        ],
        out_shape=[
            jax.ShapeDtypeStruct((_T, _K), jnp.int32),
            jax.ShapeDtypeStruct((_T, _K), jnp.float32),
            jax.ShapeDtypeStruct((_T, _K), jnp.int32),
            jax.ShapeDtypeStruct((1, _E), jnp.int32),
            jax.ShapeDtypeStruct((1, 1), jnp.float32),
        ],
        scratch_shapes=[
            pltpu.VMEM((1, _E), jnp.float32),
            pltpu.VMEM((1, _E), jnp.float32),
        ],
    )(x_flat, gate_W, gate_b)

    c = counts[0]
    cpad = ((c + (_TM - 1)) // _TM) * _TM
    base_incl = jnp.cumsum(cpad)
    tile_start = jnp.arange(_NT3, dtype=jnp.int32) * _TM
    eid = jnp.sum(tile_start[:, None] >= base_incl[None, :], axis=1)
    eid = jnp.minimum(eid, _E - 1).astype(jnp.int32)
    ntiles = (base_incl[_E - 1] // _TM).astype(jnp.int32)
    eid = jnp.concatenate([eid, ntiles[None]])

    idx_km = idx.T.reshape(_PAIRS // _SUB, _SUB)
    rank_km = rank.T.reshape(_PAIRS // _SUB, _SUB)

    slot = pl.pallas_call(
        _slot_kernel,
        in_specs=[
            pl.BlockSpec(memory_space=pltpu.SMEM),
            pl.BlockSpec((_PAIRS // _SUB, _SUB), lambda: (0, 0)),
            pl.BlockSpec((_PAIRS // _SUB, _SUB), lambda: (0, 0)),
        ],
        out_specs=pl.BlockSpec((_PAIRS // _SUB, _SUB), lambda: (0, 0)),
        out_shape=jax.ShapeDtypeStruct((_PAIRS // _SUB, _SUB), jnp.int32),
    )(counts, idx_km, rank_km)

    mesh = plsc.VectorSubcoreMesh(core_axis_name="c", subcore_axis_name="s")
    xg = pl.kernel(
        _dispatch_body,
        out_type=jax.ShapeDtypeStruct((_P, _D), jnp.float32),
        mesh=mesh,
        scratch_types=[
            pltpu.VMEM((_NSUB, _SUB), jnp.int32),
            pltpu.VMEM((_NSUB, _SUB), jnp.int32),
            pltpu.VMEM((_SUB, _D), jnp.float32),
            pltpu.SemaphoreType.DMA,
            pltpu.SemaphoreType.DMA,
        ],
    )(slot, x_flat)

    yb = pl.pallas_call(
        _ffn_kernel,
        grid_spec=pltpu.PrefetchScalarGridSpec(
            num_scalar_prefetch=1,
            grid=(_NT3,),
            in_specs=[
                pl.BlockSpec((_TM, _D), lambda i, eid: (i, 0)),
                pl.BlockSpec((1, _D, _FF), lambda i, eid: (eid[i], 0, 0)),
                pl.BlockSpec((1, 1, _FF), lambda i, eid: (eid[i], 0, 0)),
                pl.BlockSpec((1, _FF, _D), lambda i, eid: (eid[i], 0, 0)),
                pl.BlockSpec((1, 1, _D), lambda i, eid: (eid[i], 0, 0)),
            ],
            out_specs=pl.BlockSpec((_TM, _D), lambda i, eid: (i, 0)),
        ),
        out_shape=jax.ShapeDtypeStruct((_P, _D), jnp.float32),
    )(eid, xg, W1, b1.reshape(_E, 1, _FF), W2, b2.reshape(_E, 1, _D))

    y2 = pl.kernel(
        _gather_body,
        out_type=jax.ShapeDtypeStruct((_K, _T, _D), jnp.float32),
        mesh=plsc.VectorSubcoreMesh(core_axis_name="c", subcore_axis_name="s"),
        scratch_types=[
            pltpu.VMEM((_TOK_W // _SUB, _SUB), jnp.int32),
            pltpu.VMEM((_SUB, _D), jnp.float32),
            pltpu.SemaphoreType.DMA,
        ],
    )(slot, yb)

    out = pl.pallas_call(
        _combine_kernel,
        grid=(_T // _GT,),
        in_specs=[
            pl.BlockSpec((_K, _GT, _D), lambda i: (0, i, 0)),
            pl.BlockSpec((_GT, _K), lambda i: (i, 0)),
        ],
        out_specs=pl.BlockSpec((_GT, _D), lambda i: (i, 0)),
        out_shape=jax.ShapeDtypeStruct((_T, _D), jnp.float32),
    )(y2, w)

    return out.reshape(_B, _S, _D), aux[0, 0]


# speedup vs baseline: 1.3859x; 1.0020x over previous
import functools
import jax
import jax.numpy as jnp
from jax import lax
from jax.experimental import pallas as pl
from jax.experimental.pallas import tpu as pltpu
from jax.experimental.pallas import tpu_sc as plsc

_B, _S, _D, _E, _FF, _K = 2, 2048, 1024, 8, 2048, 2
_T = _B * _S
_PAIRS = _T * _K
_TM = 256
_P = _PAIRS + _E * _TM
_NT3 = _P // _TM
_GT = 512

_NC, _NS, _L = 2, 16, 16
_NW = _NC * _NS
_PER_W = _PAIRS // _NW
_SUB = 64
_NSUB = _PER_W // _SUB
_TOK_W = _T // _NW


def _gelu_tanh(x):
    t = x * x
    inner = x * (0.7978845608028654 + 0.035677408136300125 * t)
    return 0.5 * x * (1.0 + jnp.tanh(inner))


def _gate_kernel(x_ref, gw_ref, gb_ref, idx_ref, w_ref, rank_ref, counts_ref,
                 aux_ref, carry_ref, psum_ref):
    i = pl.program_id(0)

    @pl.when(i == 0)
    def _():
        carry_ref[...] = jnp.zeros_like(carry_ref)
        psum_ref[...] = jnp.zeros_like(psum_ref)

    x = x_ref[...]
    logits = jax.lax.dot_general(x, gw_ref[...], (((1,), (0,)), ((), ())),
                                 preferred_element_type=jnp.float32)
    logits = logits + gb_ref[...][None, :]
    m = jnp.max(logits, axis=-1, keepdims=True)
    ex = jnp.exp(logits - m)
    probs = ex / jnp.sum(ex, axis=-1, keepdims=True)

    e_iota = jax.lax.broadcasted_iota(jnp.int32, probs.shape, 1)
    m1 = jnp.max(probs, axis=-1, keepdims=True)
    i1 = jnp.min(jnp.where(probs >= m1, e_iota, _E), axis=-1, keepdims=True)
    oh1 = e_iota == i1
    p2 = jnp.where(oh1, -1.0, probs)
    m2 = jnp.max(p2, axis=-1, keepdims=True)
    i2 = jnp.min(jnp.where(p2 >= m2, e_iota, _E), axis=-1, keepdims=True)
    oh2 = e_iota == i2

    denom = m1 + m2
    idx_ref[...] = jnp.concatenate([i1, i2], axis=1)
    w_ref[...] = jnp.concatenate([m1 / denom, m2 / denom], axis=1)

    ohc = jnp.where(oh1 | oh2, 1.0, 0.0)
    r_iota = jax.lax.broadcasted_iota(jnp.int32, (_GT, _GT), 0)
    c_iota = jax.lax.broadcasted_iota(jnp.int32, (_GT, _GT), 1)
    lstrict = jnp.where(c_iota < r_iota, 1.0, 0.0)
    c_excl = jax.lax.dot_general(lstrict, ohc, (((1,), (0,)), ((), ())),
                                 preferred_element_type=jnp.float32)
    c_tot = carry_ref[...] + c_excl
    r1 = jnp.sum(jnp.where(oh1, c_tot, 0.0), axis=-1, keepdims=True)
    r2 = jnp.sum(jnp.where(oh2, c_tot, 0.0), axis=-1, keepdims=True)
    rank_ref[...] = jnp.concatenate([r1, r2], axis=1).astype(jnp.int32)

    carry = carry_ref[...] + jnp.sum(ohc, axis=0, keepdims=True)
    psum = psum_ref[...] + jnp.sum(probs, axis=0, keepdims=True)
    carry_ref[...] = carry
    psum_ref[...] = psum

    counts_ref[...] = carry.astype(jnp.int32)
    aux = _E * jnp.sum((carry / (_T * _K)) * (psum / _T))
    aux_ref[...] = jnp.broadcast_to(aux, (1, 1))


def _slot_kernel(counts_ref, idx_ref, rank_ref, slot_ref):
    idx = idx_ref[...]
    slot = rank_ref[...]
    base = 0
    for e in range(_E):
        slot = jnp.where(idx == e, slot + base, slot)
        ce = counts_ref[0, e]
        base = base + ((ce + (_TM - 1)) // _TM) * _TM
    slot_ref[...] = slot


def _dispatch_body(slot_hbm, x_hbm, xg_hbm, tok_v, slot_v, rows_v,
                   sem1, sem2):
    wid = lax.axis_index("s") * _NC + lax.axis_index("c")
    base = wid * _PER_W
    pltpu.sync_copy(slot_hbm.at[pl.ds(wid * _NSUB, _NSUB)], slot_v)
    for j in range(_PER_W // _L):
        r, c = j // (_SUB // _L), (j % (_SUB // _L)) * _L
        pvec = base + j * _L + lax.iota(jnp.int32, _L)
        tok_v[r, pl.ds(c, _L)] = pvec & (_T - 1)
    for s in range(_NSUB):
        pltpu.async_copy(x_hbm.at[tok_v.at[s]], rows_v, sem1).wait()
        pltpu.async_copy(rows_v, xg_hbm.at[slot_v.at[s]], sem2).wait()


def _ffn_kernel(eid_ref, xg_ref, w1_ref, b1_ref, w2_ref, b2_ref, yb_ref):
    @pl.when(pl.program_id(0) < eid_ref[_NT3])
    def _():
        h = jax.lax.dot_general(xg_ref[...], w1_ref[0],
                                (((1,), (0,)), ((), ())),
                                preferred_element_type=jnp.float32)
        h = _gelu_tanh(h + b1_ref[0])
        y = jax.lax.dot_general(h, w2_ref[0], (((1,), (0,)), ((), ())),
                                preferred_element_type=jnp.float32)
        yb_ref[...] = y + b2_ref[0]


def _gather_body(slot_hbm, yb_hbm, y2_hbm, sidx_v, rows_v, sem1):
    wid = lax.axis_index("s") * _NC + lax.axis_index("c")
    tb = wid * _TOK_W
    for k in range(_K):
        pltpu.sync_copy(
            slot_hbm.at[pl.ds(k * (_T // _SUB) + wid * (_TOK_W // _SUB),
                              _TOK_W // _SUB)], sidx_v)
        for s in range(_TOK_W // _SUB):
            pltpu.async_copy(yb_hbm.at[sidx_v.at[s]], rows_v, sem1).wait()
            pltpu.sync_copy(rows_v, y2_hbm.at[k, pl.ds(tb + s * _SUB, _SUB)])


def _combine_kernel(y2_ref, w_ref, out_ref):
    w = w_ref[...]
    out_ref[...] = w[:, 0:1] * y2_ref[0] + w[:, 1:2] * y2_ref[1]


def kernel(x, gate_W, gate_b, W1, b1, W2, b2):
    x_flat = x.reshape(_T, _D)

    nt = _T // _GT
    idx, w, rank, counts, aux = pl.pallas_call(
        _gate_kernel,
        grid=(nt,),
        in_specs=[
            pl.BlockSpec((_GT, _D), lambda i: (i, 0)),
            pl.BlockSpec((_D, _E), lambda i: (0, 0)),
            pl.BlockSpec((_E,), lambda i: (0,)),
        ],
        out_specs=[
            pl.BlockSpec((_GT, _K), lambda i: (i, 0)),
            pl.BlockSpec((_GT, _K), lambda i: (i, 0)),
            pl.BlockSpec((_GT, _K), lambda i: (i, 0)),
            pl.BlockSpec((1, _E), lambda i: (0, 0)),
            pl.BlockSpec((1, 1), lambda i: (0, 0)),
        ],
        out_shape=[
            jax.ShapeDtypeStruct((_T, _K), jnp.int32),
            jax.ShapeDtypeStruct((_T, _K), jnp.float32),
            jax.ShapeDtypeStruct((_T, _K), jnp.int32),
            jax.ShapeDtypeStruct((1, _E), jnp.int32),
            jax.ShapeDtypeStruct((1, 1), jnp.float32),
        ],
        scratch_shapes=[
            pltpu.VMEM((1, _E), jnp.float32),
            pltpu.VMEM((1, _E), jnp.float32),
        ],
    )(x_flat, gate_W, gate_b)

    c = counts[0]
    cpad = ((c + (_TM - 1)) // _TM) * _TM
    base_incl = jnp.cumsum(cpad)
    tile_start = jnp.arange(_NT3, dtype=jnp.int32) * _TM
    eid = jnp.sum(tile_start[:, None] >= base_incl[None, :], axis=1)
    eid = jnp.minimum(eid, _E - 1).astype(jnp.int32)
    ntiles = (base_incl[_E - 1] // _TM).astype(jnp.int32)
    eid = jnp.concatenate([eid, ntiles[None]])

    idx_km = idx.T.reshape(_PAIRS // _SUB, _SUB)
    rank_km = rank.T.reshape(_PAIRS // _SUB, _SUB)

    slot = pl.pallas_call(
        _slot_kernel,
        in_specs=[
            pl.BlockSpec(memory_space=pltpu.SMEM),
            pl.BlockSpec((_PAIRS // _SUB, _SUB), lambda: (0, 0)),
            pl.BlockSpec((_PAIRS // _SUB, _SUB), lambda: (0, 0)),
        ],
        out_specs=pl.BlockSpec((_PAIRS // _SUB, _SUB), lambda: (0, 0)),
        out_shape=jax.ShapeDtypeStruct((_PAIRS // _SUB, _SUB), jnp.int32),
    )(counts, idx_km, rank_km)

    mesh = plsc.VectorSubcoreMesh(core_axis_name="c", subcore_axis_name="s")
    xg = pl.kernel(
        _dispatch_body,
        out_type=jax.ShapeDtypeStruct((_P, _D), jnp.float32),
        mesh=mesh,
        scratch_types=[
            pltpu.VMEM((_NSUB, _SUB), jnp.int32),
            pltpu.VMEM((_NSUB, _SUB), jnp.int32),
            pltpu.VMEM((_SUB, _D), jnp.float32),
            pltpu.SemaphoreType.DMA,
            pltpu.SemaphoreType.DMA,
        ],
    )(slot, x_flat)

    yb = pl.pallas_call(
        _ffn_kernel,
        grid_spec=pltpu.PrefetchScalarGridSpec(
            num_scalar_prefetch=1,
            grid=(_NT3,),
            in_specs=[
                pl.BlockSpec((_TM, _D), lambda i, eid: (i, 0)),
                pl.BlockSpec((1, _D, _FF), lambda i, eid: (eid[i], 0, 0)),
                pl.BlockSpec((1, 1, _FF), lambda i, eid: (eid[i], 0, 0)),
                pl.BlockSpec((1, _FF, _D), lambda i, eid: (eid[i], 0, 0)),
                pl.BlockSpec((1, 1, _D), lambda i, eid: (eid[i], 0, 0)),
            ],
            out_specs=pl.BlockSpec((_TM, _D), lambda i, eid: (i, 0)),
        ),
        out_shape=jax.ShapeDtypeStruct((_P, _D), jnp.float32),
    )(eid, xg, W1, b1.reshape(_E, 1, _FF), W2, b2.reshape(_E, 1, _D))

    y2 = pl.kernel(
        _gather_body,
        out_type=jax.ShapeDtypeStruct((_K, _T, _D), jnp.float32),
        mesh=plsc.VectorSubcoreMesh(core_axis_name="c", subcore_axis_name="s"),
        scratch_types=[
            pltpu.VMEM((_TOK_W // _SUB, _SUB), jnp.int32),
            pltpu.VMEM((_SUB, _D), jnp.float32),
            pltpu.SemaphoreType.DMA,
        ],
    )(slot, yb)

    out = pl.pallas_call(
        _combine_kernel,
        grid=(_T // _GT,),
        in_specs=[
            pl.BlockSpec((_K, _GT, _D), lambda i: (0, i, 0)),
            pl.BlockSpec((_GT, _K), lambda i: (i, 0)),
        ],
        out_specs=pl.BlockSpec((_GT, _D), lambda i: (i, 0)),
        out_shape=jax.ShapeDtypeStruct((_T, _D), jnp.float32),
    )(y2, w)

    return out.reshape(_B, _S, _D), aux[0, 0]
